# baseline (speedup 1.0000x reference)
import functools

import jax
import jax.numpy as jnp
from jax import lax
from jax.experimental import pallas as pl
from jax.experimental.pallas import tpu as pltpu
from jax.experimental.pallas import tpu_sc as plsc

_VOCAB = 1000000
_DIM = 32
_BATCH = 16384
_L = 16

_NC = 2
_NS = 16
_NW = _NC * _NS
_BPW = _BATCH // _NW


_GDN = lax.GatherDimensionNumbers(
    offset_dims=(), collapsed_slice_dims=(0,), start_index_map=(0,))


def _shuffle(v, perm):
    return lax.gather(v, perm[:, None], _GDN, slice_sizes=(1,),
                      mode=lax.GatherScatterMode.PROMISE_IN_BOUNDS)


def _rsqrt16(x):
    i = lax.bitcast_convert_type(x, jnp.int32)
    i = jnp.int32(0x5F3759DF) - (i >> 1)
    y = lax.bitcast_convert_type(i, jnp.float32)
    half = jnp.float32(0.5)
    three_half = jnp.float32(1.5)
    for _ in range(3):
        y = y * (three_half - half * x * y * y)
    return y


def _sc_body(table_hbm, key_hbm, trace_hbm, out_hbm, idx_v, rows_v, tr_v,
             sem_g, sem_t):
    wid = lax.axis_index("s") * _NC + lax.axis_index("c")
    base = wid * _BPW

    pltpu.sync_copy(key_hbm.at[pl.ds(base, _BPW)], idx_v)
    gather = pltpu.async_copy(table_hbm.at[idx_v], rows_v, sem_g)
    tcopy = pltpu.async_copy(trace_hbm.at[pl.ds(base, _BPW)], tr_v, sem_t)
    gather.wait()
    tcopy.wait()

    def row(r, _):
        v0 = rows_v[r, 0:16] + tr_v[r, 0:16]
        v1 = rows_v[r, 16:32] + tr_v[r, 16:32]
        s = v0 * v0 + v1 * v1
        for k in (8, 4, 2, 1):
            perm = lax.iota(jnp.int32, _L) ^ k
            s = s + _shuffle(s, perm)
        rs = _rsqrt16(s + jnp.float32(1e-30))
        rows_v[r, 0:16] = v0 * rs
        rows_v[r, 16:32] = v1 * rs
        return _

    lax.fori_loop(0, _BPW, row, None)
    pltpu.sync_copy(rows_v, out_hbm.at[pl.ds(base, _BPW)])


@jax.jit
def _sc_call(table, key, trace):
    mesh = plsc.VectorSubcoreMesh(core_axis_name="c", subcore_axis_name="s")
    f = pl.kernel(
        _sc_body,
        out_type=jax.ShapeDtypeStruct((_BATCH, _DIM), jnp.float32),
        mesh=mesh,
        scratch_types=[
            pltpu.VMEM((_BPW,), jnp.int32),
            pltpu.VMEM((_BPW, _DIM), jnp.float32),
            pltpu.VMEM((_BPW, _DIM), jnp.float32),
            pltpu.SemaphoreType.DMA,
            pltpu.SemaphoreType.DMA,
        ],
        compiler_params=pltpu.CompilerParams(use_tc_tiling_on_sc=False),
    )
    return f(table, key, trace)


def kernel(arg0_unused, trace, arg2_unused, key, table):
    return _sc_call(table, key.astype(jnp.int32), trace)
